# baseline (speedup 1.0000x reference)
import jax
import jax.numpy as jnp
import numpy as np
from jax.experimental import pallas as pl
from jax.experimental.pallas import tpu as pltpu

_ALPHA = 0.5
_LAM = float(np.random.RandomState(0).beta(_ALPHA, 1.0 - _ALPHA))

_R = 8
_SB = 392


def _mix_body(idx_ref, x_ref, *refs):
    g_refs = refs[:_R]
    o_ref = refs[_R]
    for r in range(_R):
        o_ref[r] = _LAM * x_ref[r] + (1.0 - _LAM) * g_refs[r][0]


def kernel(x, y, index):
    B = x.shape[0]
    S = 1176
    x3 = x.reshape(B, S, 128)

    def _gspec(r):
        return pl.BlockSpec(
            (1, _SB, 128), lambda i, j, idx, r=r: (idx[_R * i + r], j, 0))

    out = pl.pallas_call(
        _mix_body,
        grid_spec=pltpu.PrefetchScalarGridSpec(
            num_scalar_prefetch=1,
            grid=(B // _R, S // _SB),
            in_specs=[pl.BlockSpec((_R, _SB, 128), lambda i, j, idx: (i, j, 0))]
            + [_gspec(r) for r in range(_R)],
            out_specs=pl.BlockSpec((_R, _SB, 128), lambda i, j, idx: (i, j, 0)),
        ),
        out_shape=jax.ShapeDtypeStruct((B, S, 128), jnp.float32),
    )(index, x3, *([x3] * _R))
    mixed = out.reshape(x.shape)
    y_b = jnp.take(y, index, axis=0)
    return (mixed, y, y_b, jnp.float32(_LAM))
